# Initial kernel scaffold; baseline (speedup 1.0000x reference)
import jax
import jax.numpy as jnp
from jax.experimental import pallas as pl


def kernel(user_ids, item_ids, edge_user, edge_item, user_emb, item_emb):
    raise NotImplementedError("write your pallas kernel here")


# speedup vs baseline: 1.0320x; 1.0320x over previous
import functools
import jax
import jax.numpy as jnp
from jax import lax
from jax.experimental import pallas as pl
from jax.experimental.pallas import tpu as pltpu
from jax.experimental.pallas import tpu_sc as plsc

N_USER = 50000
N_ITEM = 50000
N_TOTAL = N_USER + N_ITEM
F = 64
GCN_LAYER = 3
N_EDGES = 800000

NW = 32
FP = 128
CHUNK_E = 128
STEPS = 392
E_PAD = NW * STEPS * CHUNK_E
NODE_CHUNK = 12500
TBL_ROWS = 12544
N_PASS = 8
ROWS_PER_SUB = TBL_ROWS // 16


def _make_segsum():
    mesh = plsc.VectorSubcoreMesh(core_axis_name="c", subcore_axis_name="s")

    @functools.partial(
        pl.kernel, mesh=mesh,
        out_type=jax.ShapeDtypeStruct((2, N_PASS * TBL_ROWS, FP), jnp.float32),
        scratch_types=[
            pltpu.VMEM((CHUNK_E,), jnp.int32),
            pltpu.VMEM((CHUNK_E,), jnp.int32),
            pltpu.VMEM((CHUNK_E,), jnp.int32),
            pltpu.VMEM((CHUNK_E, FP), jnp.float32),
            pltpu.VMEM_SHARED((TBL_ROWS, FP), jnp.float32),
            pltpu.SemaphoreType.DMA,
        ],
    )
    def segsum(u_hbm, cols_hbm, rows_hbm, zeros_hbm, out_hbm,
               cols_v, rows_v, idx_v, gath_v, table, sem):
        cid = lax.axis_index("c")
        sid = lax.axis_index("s")
        wid = sid * 2 + cid

        def one_pass(p, _):
            lo = p * NODE_CHUNK
            pltpu.sync_copy(
                zeros_hbm.at[pl.ds(sid * ROWS_PER_SUB, ROWS_PER_SUB)],
                table.at[pl.ds(sid * ROWS_PER_SUB, ROWS_PER_SUB)])
            plsc.subcore_barrier()

            def one_step(j, _):
                base = wid * (STEPS * CHUNK_E) + j * CHUNK_E
                pltpu.sync_copy(cols_hbm.at[pl.ds(base, CHUNK_E)], cols_v)
                pltpu.sync_copy(rows_hbm.at[pl.ds(base, CHUNK_E)], rows_v)
                for i in range(CHUNK_E // 16):
                    r = rows_v[pl.ds(i * 16, 16)]
                    ok = (r >= lo) & (r < lo + NODE_CHUNK)
                    idx_v[pl.ds(i * 16, 16)] = jnp.where(ok, r - lo, NODE_CHUNK)
                pltpu.async_copy(u_hbm.at[cols_v], gath_v, sem).wait()
                pltpu.sync_copy(gath_v, table.at[idx_v], add=True)
                return 0

            lax.fori_loop(0, STEPS, one_step, 0)
            plsc.subcore_barrier()
            pltpu.sync_copy(
                table.at[pl.ds(sid * ROWS_PER_SUB, ROWS_PER_SUB)],
                out_hbm.at[cid].at[pl.ds(p * TBL_ROWS + sid * ROWS_PER_SUB,
                                         ROWS_PER_SUB)])
            plsc.subcore_barrier()
            return 0

        lax.fori_loop(0, N_PASS, one_pass, 0)

    return segsum


def _make_pairgather():
    mesh = plsc.VectorSubcoreMesh(core_axis_name="c", subcore_axis_name="s")
    B = 8192
    bpw = B // NW

    @functools.partial(
        pl.kernel, mesh=mesh,
        out_type=jax.ShapeDtypeStruct((B, FP), jnp.float32),
        scratch_types=[
            pltpu.VMEM((bpw,), jnp.int32),
            pltpu.VMEM((bpw, FP), jnp.float32),
            pltpu.SemaphoreType.DMA,
        ],
    )
    def pairgather(table_hbm, idx_hbm, out_hbm, idx_v, rows_v, sem):
        wid = lax.axis_index("s") * 2 + lax.axis_index("c")
        base = wid * bpw
        pltpu.sync_copy(idx_hbm.at[pl.ds(base, bpw)], idx_v)
        pltpu.async_copy(table_hbm.at[idx_v], rows_v, sem).wait()
        pltpu.sync_copy(rows_v, out_hbm.at[pl.ds(base, bpw)])

    return pairgather


_BLK = 4000
_GRID = N_TOTAL // _BLK
_espec = pl.BlockSpec((_BLK, F), lambda i: (i, 0))


def _tc_eltwise(body, n_out, *args):
    outs = tuple(jax.ShapeDtypeStruct((N_TOTAL, F), jnp.float32)
                 for _ in range(n_out))
    return pl.pallas_call(
        body, grid=(_GRID,),
        in_specs=[_espec] * len(args),
        out_specs=(_espec,) * n_out if n_out > 1 else _espec,
        out_shape=outs if n_out > 1 else outs[0],
    )(*args)


def _dinv_body(p0, p1, dinv):
    deg = p0[...] + p1[...] + 1e-8
    dinv[...] = lax.rsqrt(deg)


def _layer_body(p0, p1, dinv, acc_in, emb, u, acc_out):
    e = dinv[...] * (p0[...] + p1[...])
    emb[...] = e
    u[...] = dinv[...] * e
    acc_out[...] = acc_in[...] + e


def _dot_body(gu, gi, out):
    out[...] = jnp.sum(gu[...] * gi[...], axis=1) * (1.0 / ((GCN_LAYER + 1) ** 2))


def _combine(partials):
    p = partials.reshape(2, N_PASS, TBL_ROWS, FP)[:, :, :NODE_CHUNK, :F]
    p = p.reshape(2, N_TOTAL, F)
    return p[0], p[1]


def _fpad(x):
    return jnp.pad(x, ((0, 0), (0, FP - F)))


def kernel(user_ids, item_ids, edge_user, edge_item, user_emb, item_emb):
    segsum = _make_segsum()
    pairgather = _make_pairgather()

    u = edge_user
    i = edge_item + N_USER
    rows = jnp.concatenate([u, i])
    cols = jnp.concatenate([i, u])
    pad = E_PAD - 2 * N_EDGES
    rows = jnp.concatenate([rows, jnp.full((pad,), 1 << 20, jnp.int32)])
    cols = jnp.concatenate([cols, jnp.zeros((pad,), jnp.int32)])
    zeros = jnp.zeros((TBL_ROWS, FP), jnp.float32)
    ones_tbl = jnp.ones((N_TOTAL, FP), jnp.float32)

    d0, d1 = _combine(segsum(ones_tbl, cols, rows, zeros))
    dinv = _tc_eltwise(_dinv_body, 1, d0, d1)

    ego = jnp.concatenate([user_emb, item_emb], axis=0)
    u_tbl = dinv * ego
    acc = ego
    for _ in range(GCN_LAYER):
        p0, p1 = _combine(segsum(_fpad(u_tbl), cols, rows, zeros))
        _emb, u_tbl, acc = _tc_eltwise(_layer_body, 3, p0, p1, dinv, acc)

    idx = jnp.concatenate([user_ids, item_ids + N_USER])
    g = pairgather(_fpad(acc), idx)
    gu, gi = g[:4096, :F], g[4096:, :F]
    preds = pl.pallas_call(
        _dot_body,
        out_shape=jax.ShapeDtypeStruct((4096,), jnp.float32),
    )(gu, gi)
    return preds
